# baseline (speedup 1.0000x reference)
import jax
import jax.numpy as jnp
from jax import lax
from jax.experimental import pallas as pl
from jax.experimental.pallas import tpu as pltpu
from jax.experimental.pallas import tpu_sc as plsc
from jax._src.pallas import core as _pallas_core
from jax._src.pallas.mosaic import core as _tpu_core

_N, _C = 65536, 256

_NW = 32
_TEC_ROWS = 40960
_TEC_PER_W = _TEC_ROWS // _NW
_TR = 184

_NSCS = 2
_SCS_ROWS = _N - _TEC_ROWS
_SCS_PER_W = _SCS_ROWS // _NSCS
_SR = 1024


def _chunk_list(total, step):
    return [(o, min(step, total - o)) for o in range(0, total, step)]

_TCH = _chunk_list(_TEC_PER_W, _TR)
_SCH = _chunk_list(_SCS_PER_W, _SR)


def _ring(x_hbm, out_hbm, base, chunks, bufs, lds, sts, patch_wid=None):
    def ld_copy(idx):
        off, ln = chunks[idx]
        b = idx % 2
        return pltpu.make_async_copy(
            x_hbm.at[pl.ds(base + off, ln)],
            bufs[b].at[pl.ds(0, ln)], lds[b])

    def st_copies(idx):
        off, ln = chunks[idx]
        b = idx % 2
        return (
            pltpu.make_async_copy(
                bufs[b].at[pl.ds(0, ln)],
                out_hbm.at[pl.ds(base + off, ln)], sts[b]),
            pltpu.make_async_copy(
                bufs[b].at[pl.ds(0, ln)],
                out_hbm.at[pl.ds(_N + base + off, ln)], sts[b]),
        )

    ld_copy(0).start()
    if len(chunks) > 1:
        ld_copy(1).start()

    for idx in range(len(chunks)):
        ld_copy(idx).wait()

        if idx == 0 and patch_wid is not None:
            @pl.when(patch_wid == 0)
            def _patch():
                v = bufs[0][0, pl.ds(0, 16)]
                lane = lax.iota(jnp.int32, 16)
                bufs[0][0, pl.ds(0, 16)] = jnp.where(
                    lane == 0, jnp.float32(100.0), v)

        s1, s2 = st_copies(idx)
        s1.start()
        s2.start()

        if idx + 2 < len(chunks):
            s1.wait()
            s2.wait()
            ld_copy(idx + 2).start()

    for idx in range(max(0, len(chunks) - 2), len(chunks)):
        s1, s2 = st_copies(idx)
        s1.wait()
        s2.wait()


def _tec_body(x_hbm, out_hbm, tb0, tb1, sb0, sb1):
    wid = lax.axis_index("s") * 2 + lax.axis_index("c")
    base = wid * _TEC_PER_W

    def scoped(ld0, ld1, st0, st1):
        _ring(x_hbm, out_hbm, base, _TCH, (tb0, tb1), (ld0, ld1),
              (st0, st1), patch_wid=wid)

    pl.run_scoped(
        scoped,
        pltpu.SemaphoreType.DMA,
        pltpu.SemaphoreType.DMA,
        pltpu.SemaphoreType.DMA,
        pltpu.SemaphoreType.DMA,
    )


def _scs_body(x_hbm, out_hbm, tb0, tb1, sb0, sb1):
    cid = lax.axis_index("c")
    base = _TEC_ROWS + cid * _SCS_PER_W

    def scoped(ld0, ld1, st0, st1):
        _ring(x_hbm, out_hbm, base, _SCH, (sb0, sb1), (ld0, ld1),
              (st0, st1))

    pl.run_scoped(
        scoped,
        pltpu.SemaphoreType.DMA,
        pltpu.SemaphoreType.DMA,
        pltpu.SemaphoreType.DMA,
        pltpu.SemaphoreType.DMA,
    )


def kernel(x):
    v_mesh = plsc.VectorSubcoreMesh(core_axis_name="c",
                                    subcore_axis_name="s")
    s_mesh = plsc.ScalarSubcoreMesh(axis_name="c", num_cores=_NSCS)
    tec_vmem = _pallas_core.CoreMemorySpace(
        _tpu_core.MemorySpace.VMEM, v_mesh)
    f = pl.kernel(
        body=[_tec_body, _scs_body],
        mesh=[v_mesh, s_mesh],
        out_type=jax.ShapeDtypeStruct((2 * _N, _C), jnp.float32),
        scratch_types=[
            tec_vmem((_TR, _C), jnp.float32),
            tec_vmem((_TR, _C), jnp.float32),
            pltpu.VMEM_SHARED((_SR, _C), jnp.float32),
            pltpu.VMEM_SHARED((_SR, _C), jnp.float32),
        ],
    )
    return f(x)


# speedup vs baseline: 1.0181x; 1.0181x over previous
import jax
import jax.numpy as jnp
from jax import lax
from jax.experimental import pallas as pl
from jax.experimental.pallas import tpu as pltpu
from jax.experimental.pallas import tpu_sc as plsc

_N, _C = 65536, 256
_BM = 2048


def _tc_copy_body(x_ref, o_ref):
    v = x_ref[...]
    o_ref[0] = v
    o_ref[1] = v


def _sc_patch_body(y_ref):
    wid = lax.axis_index("s") * 2 + lax.axis_index("c")

    def scoped(buf, sem):
        @pl.when(wid == 0)
        def _():
            pltpu.async_copy(y_ref.at[0, pl.ds(0, 16)], buf, sem).wait()
            lane = lax.iota(jnp.int32, 16)
            buf[...] = jnp.where(lane == 0, jnp.float32(100.0), buf[...])
            c1 = pltpu.async_copy(buf, y_ref.at[0, pl.ds(0, 16)], sem)
            c2 = pltpu.async_copy(buf, y_ref.at[_N, pl.ds(0, 16)], sem)
            c1.wait()
            c2.wait()

    pl.run_scoped(scoped, pltpu.VMEM((16,), jnp.float32),
                  pltpu.SemaphoreType.DMA)


def kernel(x):
    n, c = x.shape
    raw = pl.pallas_call(
        _tc_copy_body,
        grid=(n // _BM,),
        in_specs=[pl.BlockSpec((_BM, c), lambda i: (i, 0))],
        out_specs=pl.BlockSpec((2, _BM, c), lambda i: (0, i, 0)),
        out_shape=jax.ShapeDtypeStruct((2, n, c), x.dtype),
    )(x).reshape(2 * n, c)

    v_mesh = plsc.VectorSubcoreMesh(core_axis_name="c",
                                    subcore_axis_name="s")
    y_ref = jax.new_ref(raw)
    patch = pl.kernel(_sc_patch_body, mesh=v_mesh)
    patch(y_ref)
    return jax.freeze(y_ref)


# speedup vs baseline: 1.0361x; 1.0177x over previous
import jax
import jax.numpy as jnp
from jax import lax
from jax.experimental import pallas as pl
from jax.experimental.pallas import tpu as pltpu
from jax.experimental.pallas import tpu_sc as plsc

_N, _C = 65536, 256
_BM = 2048


def _tc_copy_body(x_ref, o_ref):
    v = x_ref[...]
    o_ref[0] = v
    o_ref[1] = v


def _sc_patch_body(y_ref):
    wid = lax.axis_index("s") * 2 + lax.axis_index("c")

    def scoped(buf, sem):
        @pl.when(wid == 0)
        def _():
            pltpu.async_copy(y_ref.at[0, pl.ds(0, 16)], buf, sem).wait()
            lane = lax.iota(jnp.int32, 16)
            buf[...] = jnp.where(lane == 0, jnp.float32(100.0), buf[...])
            c1 = pltpu.async_copy(buf, y_ref.at[0, pl.ds(0, 16)], sem)
            c2 = pltpu.async_copy(buf, y_ref.at[_N, pl.ds(0, 16)], sem)
            c1.wait()
            c2.wait()

    pl.run_scoped(scoped, pltpu.VMEM((16,), jnp.float32),
                  pltpu.SemaphoreType.DMA)


def kernel(x):
    n, c = x.shape
    raw = pl.pallas_call(
        _tc_copy_body,
        grid=(n // _BM,),
        in_specs=[pl.BlockSpec((_BM, c), lambda i: (i, 0))],
        out_specs=pl.BlockSpec((2, _BM, c), lambda i: (0, i, 0)),
        out_shape=jax.ShapeDtypeStruct((2, n, c), x.dtype),
    )(x).reshape(2 * n, c)

    v_mesh = plsc.VectorSubcoreMesh(core_axis_name="c",
                                    subcore_axis_name="s",
                                    num_cores=1, num_subcores=1)
    y_ref = jax.new_ref(raw)
    patch = pl.kernel(_sc_patch_body, mesh=v_mesh)
    patch(y_ref)
    return jax.freeze(y_ref)
